# baseline (speedup 1.0000x reference)
import functools

import jax
import jax.numpy as jnp
from jax.experimental import pallas as pl
from jax.experimental.pallas import tpu as pltpu

_N = 4096
_SEQ = 8
_K = 32
_R_KNN2 = 0.25
_R_BQ2 = 0.5625
_BIG = 1e30
_ROWS = 256


def _tile_body(p4_ref, pt_ref, f_ref, ft_ref, out_ref, wsum_ref):
    i = pl.program_id(0)
    rows = _ROWS

    pr = p4_ref[:, 0:3]
    w = p4_ref[:, 3:4]
    pt = pt_ref[...]

    colg = jax.lax.broadcasted_iota(jnp.int32, (rows, _N), 1)
    rowg = i * rows + jax.lax.broadcasted_iota(jnp.int32, (rows, _N), 0)
    diag = colg == rowg

    p2r = jnp.sum(pr * pr, axis=1, keepdims=True)
    p2c = jnp.sum(pt * pt, axis=0, keepdims=True)
    dot = jnp.dot(pr, pt, preferred_element_type=jnp.float32,
                  precision=jax.lax.Precision.DEFAULT)
    d2 = jnp.maximum(p2r + p2c - 2.0 * dot, 0.0)

    inball = d2 < _R_BQ2
    ibf = inball.astype(jnp.float32)
    cs = ibf
    sh = 1
    while sh < _N:
        cs = cs + jnp.concatenate(
            [jnp.zeros((rows, sh), jnp.float32), cs[:, :-sh]], axis=1)
        sh *= 2
    wb = jnp.where(inball & (cs <= float(_K)), 1.0, 0.0)
    cnt = jnp.sum(ibf, axis=1, keepdims=True)
    padb = jnp.maximum(float(_K) - cnt, 0.0)

    inrad = d2 <= _R_KNN2
    bits = jax.lax.bitcast_convert_type(jnp.where(inrad, d2, _BIG),
                                        jnp.int32)
    c_r = jnp.sum(inrad.astype(jnp.float32), axis=1, keepdims=True)
    cur = jnp.zeros((rows, 1), jnp.int32)
    for b in range(30, -1, -1):
        trial = cur | (1 << b)
        cnt = jnp.sum((bits < trial).astype(jnp.float32), axis=1,
                      keepdims=True)
        cur = jnp.where(cnt < float(_K), trial, cur)
    wk = jnp.where((bits <= cur) & inrad, 1.0, 0.0)
    padk = jnp.maximum(float(_K) - c_r, 0.0)

    colf = colg.astype(jnp.float32)
    m0 = jnp.min(d2, axis=1, keepdims=True)
    iselk0 = jnp.min(jnp.where(d2 == m0, colf, float(_N)), axis=1,
                     keepdims=True)
    wt = (wk + wb
          + padk * jnp.where(colf == iselk0, 1.0, 0.0)
          + padb * jnp.where(inball & (cs == 1.0), 1.0, 0.0))
    acc = jnp.zeros((rows, 1), jnp.float32)
    for s in range(_SEQ):
        frs = f_ref[s]
        fts = ft_ref[s]
        f2r = jnp.sum(frs * frs, axis=1, keepdims=True)
        f2c = jnp.sum(fts * fts, axis=0, keepdims=True)
        fdot = jnp.dot(frs, fts, preferred_element_type=jnp.float32,
                       precision=jax.lax.Precision.HIGHEST)
        fd2 = f2r + f2c - 2.0 * fdot
        fd2 = jnp.where(diag, 0.0, fd2)
        nrm = jnp.where(fd2 > 0.0, jnp.sqrt(jnp.where(fd2 > 0.0, fd2, 1.0)),
                        0.0)
        acc = acc + jnp.sum(wt * nrm, axis=1, keepdims=True)

    @pl.when(i == 0)
    def _():
        out_ref[...] = jnp.zeros((1, 1), jnp.float32)
        wsum_ref[...] = jnp.zeros((1, 1), jnp.float32)

    out_ref[...] += jnp.sum(w * acc, axis=0, keepdims=True)
    wsum_ref[...] += jnp.sum(w, axis=0, keepdims=True)


@jax.jit
def kernel(pc_source, pred_flow, weights):
    p = pc_source[0]
    p4 = jnp.concatenate([p, weights[:, None]], axis=1)
    pt = p.T
    ft = jnp.transpose(pred_flow, (0, 2, 1))

    nb = _N // _ROWS
    total, wsum = pl.pallas_call(
        _tile_body,
        grid=(nb,),
        in_specs=[
            pl.BlockSpec((_ROWS, 4), lambda i: (i, 0)),
            pl.BlockSpec((3, _N), lambda i: (0, 0)),
            pl.BlockSpec((_SEQ, _ROWS, 3), lambda i: (0, i, 0)),
            pl.BlockSpec((_SEQ, 3, _N), lambda i: (0, 0, 0)),
        ],
        out_specs=[
            pl.BlockSpec((1, 1), lambda i: (0, 0)),
            pl.BlockSpec((1, 1), lambda i: (0, 0)),
        ],
        out_shape=[
            jax.ShapeDtypeStruct((1, 1), jnp.float32),
            jax.ShapeDtypeStruct((1, 1), jnp.float32),
        ],
        compiler_params=pltpu.CompilerParams(
            dimension_semantics=("arbitrary",),
        ),
    )(p4, pt, pred_flow, ft)

    t = total[0, 0]
    ws = wsum[0, 0]
    denom = jnp.where(ws > 0.0, ws, 1.0)
    return (t / denom) / (_K * _SEQ)


# speedup vs baseline: 1.0744x; 1.0744x over previous
import functools

import jax
import jax.numpy as jnp
from jax.experimental import pallas as pl
from jax.experimental.pallas import tpu as pltpu

_N = 4096
_SEQ = 8
_K = 32
_R_KNN2 = 0.25
_R_BQ2 = 0.5625
_BIG = 1e30
_ROWS = 256


def _tile_body(p4_ref, pt_ref, f_ref, ft_ref, out_ref, wsum_ref):
    i = pl.program_id(0)
    rows = _ROWS

    pr = p4_ref[:, 0:3]
    w = p4_ref[:, 3:4]
    pt = pt_ref[...]

    colg = jax.lax.broadcasted_iota(jnp.int32, (rows, _N), 1)
    rowg = i * rows + jax.lax.broadcasted_iota(jnp.int32, (rows, _N), 0)
    diag = colg == rowg

    p2r = jnp.sum(pr * pr, axis=1, keepdims=True)
    p2c = jnp.sum(pt * pt, axis=0, keepdims=True)
    dot = jnp.dot(pr, pt, preferred_element_type=jnp.float32,
                  precision=jax.lax.Precision.DEFAULT)
    d2 = jnp.maximum(p2r + p2c - 2.0 * dot, 0.0)

    inball = d2 < _R_BQ2
    ibf = inball.astype(jnp.float32)
    cs = ibf
    sh = 1
    while sh < _N:
        cs = cs + jnp.concatenate(
            [jnp.zeros((rows, sh), jnp.float32), cs[:, :-sh]], axis=1)
        sh *= 2
    wb = jnp.where(inball & (cs <= float(_K)), 1.0, 0.0)
    cnt = jnp.sum(ibf, axis=1, keepdims=True)
    padb = jnp.maximum(float(_K) - cnt, 0.0)

    inrad = d2 <= _R_KNN2
    bits = jax.lax.bitcast_convert_type(jnp.where(inrad, d2, _BIG),
                                        jnp.int32)
    c_r = jnp.sum(inrad.astype(jnp.float32), axis=1, keepdims=True)
    cur = jnp.zeros((rows, 1), jnp.int32)
    for b in range(30, -1, -1):
        trial = cur | (1 << b)
        cnt = jnp.sum((bits < trial).astype(jnp.float32), axis=1,
                      keepdims=True)
        cur = jnp.where(cnt < float(_K), trial, cur)
    wk = jnp.where((bits <= cur) & inrad, 1.0, 0.0)
    padk = jnp.maximum(float(_K) - c_r, 0.0)

    colf = colg.astype(jnp.float32)
    m0 = jnp.min(d2, axis=1, keepdims=True)
    iselk0 = jnp.min(jnp.where(d2 == m0, colf, float(_N)), axis=1,
                     keepdims=True)
    wt = (wk + wb
          + padk * jnp.where(colf == iselk0, 1.0, 0.0)
          + padb * jnp.where(inball & (cs == 1.0), 1.0, 0.0))
    acc = jnp.zeros((rows, 1), jnp.float32)
    for s in range(_SEQ):
        frs = f_ref[s]
        fts = ft_ref[s]
        f2r = jnp.sum(frs * frs, axis=1, keepdims=True)
        f2c = jnp.sum(fts * fts, axis=0, keepdims=True)
        fdot = jnp.dot(frs, fts, preferred_element_type=jnp.float32,
                       precision=jax.lax.Precision.DEFAULT)
        fd2 = f2r + f2c - 2.0 * fdot
        fd2 = jnp.where(diag, 0.0, fd2)
        nrm = jnp.where(fd2 > 0.0, jnp.sqrt(jnp.where(fd2 > 0.0, fd2, 1.0)),
                        0.0)
        acc = acc + jnp.sum(wt * nrm, axis=1, keepdims=True)

    @pl.when(i == 0)
    def _():
        out_ref[...] = jnp.zeros((1, 1), jnp.float32)
        wsum_ref[...] = jnp.zeros((1, 1), jnp.float32)

    out_ref[...] += jnp.sum(w * acc, axis=0, keepdims=True)
    wsum_ref[...] += jnp.sum(w, axis=0, keepdims=True)


@jax.jit
def kernel(pc_source, pred_flow, weights):
    p = pc_source[0]
    p4 = jnp.concatenate([p, weights[:, None]], axis=1)
    pt = p.T
    ft = jnp.transpose(pred_flow, (0, 2, 1))

    nb = _N // _ROWS
    total, wsum = pl.pallas_call(
        _tile_body,
        grid=(nb,),
        in_specs=[
            pl.BlockSpec((_ROWS, 4), lambda i: (i, 0)),
            pl.BlockSpec((3, _N), lambda i: (0, 0)),
            pl.BlockSpec((_SEQ, _ROWS, 3), lambda i: (0, i, 0)),
            pl.BlockSpec((_SEQ, 3, _N), lambda i: (0, 0, 0)),
        ],
        out_specs=[
            pl.BlockSpec((1, 1), lambda i: (0, 0)),
            pl.BlockSpec((1, 1), lambda i: (0, 0)),
        ],
        out_shape=[
            jax.ShapeDtypeStruct((1, 1), jnp.float32),
            jax.ShapeDtypeStruct((1, 1), jnp.float32),
        ],
        compiler_params=pltpu.CompilerParams(
            dimension_semantics=("arbitrary",),
        ),
    )(p4, pt, pred_flow, ft)

    t = total[0, 0]
    ws = wsum[0, 0]
    denom = jnp.where(ws > 0.0, ws, 1.0)
    return (t / denom) / (_K * _SEQ)


# speedup vs baseline: 1.1922x; 1.1097x over previous
import functools

import jax
import jax.numpy as jnp
from jax.experimental import pallas as pl
from jax.experimental.pallas import tpu as pltpu

_N = 4096
_SEQ = 8
_K = 32
_R_KNN2 = 0.25
_R_BQ2 = 0.5625
_BIG = 1e30
_ROWS = 256


def _tile_body(p4_ref, pt_ref, f_ref, ft_ref, out_ref, wsum_ref):
    i = pl.program_id(0)
    rows = _ROWS

    pr = p4_ref[:, 0:3]
    w = p4_ref[:, 3:4]
    pt = pt_ref[...]

    colg = jax.lax.broadcasted_iota(jnp.int32, (rows, _N), 1)
    rowg = i * rows + jax.lax.broadcasted_iota(jnp.int32, (rows, _N), 0)
    diag = colg == rowg

    p2r = jnp.sum(pr * pr, axis=1, keepdims=True)
    p2c = jnp.sum(pt * pt, axis=0, keepdims=True)
    dot = jnp.dot(pr, pt, preferred_element_type=jnp.float32,
                  precision=jax.lax.Precision.DEFAULT)
    d2 = jnp.maximum(p2r + p2c - 2.0 * dot, 0.0)

    inball = d2 < _R_BQ2
    ibf = inball.astype(jnp.float32)
    cs = ibf
    sh = 1
    while sh < _N:
        cs = cs + jnp.concatenate(
            [jnp.zeros((rows, sh), jnp.float32), cs[:, :-sh]], axis=1)
        sh *= 2
    wb = jnp.where(inball & (cs <= float(_K)), 1.0, 0.0)
    cnt = jnp.sum(ibf, axis=1, keepdims=True)
    padb = jnp.maximum(float(_K) - cnt, 0.0)

    inrad = d2 <= _R_KNN2
    bits = jax.lax.bitcast_convert_type(jnp.where(inrad, d2, _BIG),
                                        jnp.int32)
    c_r = jnp.sum(inrad.astype(jnp.float32), axis=1, keepdims=True)
    cur = jnp.zeros((rows, 1), jnp.int32)
    for b in range(30, -1, -1):
        trial = cur | (1 << b)
        cnt = jnp.sum((bits < trial).astype(jnp.float32), axis=1,
                      keepdims=True)
        cur = jnp.where(cnt < float(_K), trial, cur)
    wk = jnp.where((bits <= cur) & inrad, 1.0, 0.0)
    padk = jnp.maximum(float(_K) - c_r, 0.0)

    colf = colg.astype(jnp.float32)
    m0 = jnp.min(d2, axis=1, keepdims=True)
    iselk0 = jnp.min(jnp.where(d2 == m0, colf, float(_N)), axis=1,
                     keepdims=True)
    wt = (wk + wb
          + padk * jnp.where(colf == iselk0, 1.0, 0.0)
          + padb * jnp.where(inball & (cs == 1.0), 1.0, 0.0))
    nsum = jnp.zeros((rows, _N), jnp.float32)
    for s in range(_SEQ):
        frs = f_ref[s]
        fts = ft_ref[s]
        f2r = jnp.sum(frs * frs, axis=1, keepdims=True)
        f2c = jnp.sum(fts * fts, axis=0, keepdims=True)
        fdot = jnp.dot(frs, fts, preferred_element_type=jnp.float32,
                       precision=jax.lax.Precision.DEFAULT)
        fd2 = f2r + f2c - 2.0 * fdot
        nsum = nsum + jnp.sqrt(jnp.maximum(fd2, 0.0))
    nsum = jnp.where(diag, 0.0, nsum)
    acc = jnp.sum(wt * nsum, axis=1, keepdims=True)

    @pl.when(i == 0)
    def _():
        out_ref[...] = jnp.zeros((1, 1), jnp.float32)
        wsum_ref[...] = jnp.zeros((1, 1), jnp.float32)

    out_ref[...] += jnp.sum(w * acc, axis=0, keepdims=True)
    wsum_ref[...] += jnp.sum(w, axis=0, keepdims=True)


@jax.jit
def kernel(pc_source, pred_flow, weights):
    p = pc_source[0]
    p4 = jnp.concatenate([p, weights[:, None]], axis=1)
    pt = p.T
    ft = jnp.transpose(pred_flow, (0, 2, 1))

    nb = _N // _ROWS
    total, wsum = pl.pallas_call(
        _tile_body,
        grid=(nb,),
        in_specs=[
            pl.BlockSpec((_ROWS, 4), lambda i: (i, 0)),
            pl.BlockSpec((3, _N), lambda i: (0, 0)),
            pl.BlockSpec((_SEQ, _ROWS, 3), lambda i: (0, i, 0)),
            pl.BlockSpec((_SEQ, 3, _N), lambda i: (0, 0, 0)),
        ],
        out_specs=[
            pl.BlockSpec((1, 1), lambda i: (0, 0)),
            pl.BlockSpec((1, 1), lambda i: (0, 0)),
        ],
        out_shape=[
            jax.ShapeDtypeStruct((1, 1), jnp.float32),
            jax.ShapeDtypeStruct((1, 1), jnp.float32),
        ],
        compiler_params=pltpu.CompilerParams(
            dimension_semantics=("arbitrary",),
        ),
    )(p4, pt, pred_flow, ft)

    t = total[0, 0]
    ws = wsum[0, 0]
    denom = jnp.where(ws > 0.0, ws, 1.0)
    return (t / denom) / (_K * _SEQ)


# speedup vs baseline: 1.5363x; 1.2886x over previous
import functools

import jax
import jax.numpy as jnp
from jax.experimental import pallas as pl
from jax.experimental.pallas import tpu as pltpu

_N = 4096
_SEQ = 8
_K = 32
_R_KNN2 = 0.25
_R_BQ2 = 0.5625
_BIG = 1e30
_ROWS = 128


def _tile_body(p4_ref, pt_ref, f_ref, ft_ref, out_ref, wsum_ref):
    i = pl.program_id(0)
    rows = _ROWS

    pr = p4_ref[:, 0:3]
    w = p4_ref[:, 3:4]
    pt = pt_ref[...]

    colg = jax.lax.broadcasted_iota(jnp.int32, (rows, _N), 1)
    rowg = i * rows + jax.lax.broadcasted_iota(jnp.int32, (rows, _N), 0)
    diag = colg == rowg

    p2r = jnp.sum(pr * pr, axis=1, keepdims=True)
    p2c = jnp.sum(pt * pt, axis=0, keepdims=True)
    dot = jnp.dot(pr, pt, preferred_element_type=jnp.float32,
                  precision=jax.lax.Precision.DEFAULT)
    d2 = jnp.maximum(p2r + p2c - 2.0 * dot, 0.0)

    inball = d2 < _R_BQ2
    ibf = inball.astype(jnp.float32)
    cs = ibf
    sh = 1
    while sh < _N:
        cs = cs + jnp.concatenate(
            [jnp.zeros((rows, sh), jnp.float32), cs[:, :-sh]], axis=1)
        sh *= 2
    wb = jnp.where(inball & (cs <= float(_K)), 1.0, 0.0)
    cnt = jnp.sum(ibf, axis=1, keepdims=True)
    padb = jnp.maximum(float(_K) - cnt, 0.0)

    inrad = d2 <= _R_KNN2
    bits = jax.lax.bitcast_convert_type(jnp.where(inrad, d2, _BIG),
                                        jnp.int32)
    c_r = jnp.sum(inrad.astype(jnp.float32), axis=1, keepdims=True)
    cur = jnp.zeros((rows, 1), jnp.int32)
    for b in range(30, -1, -1):
        trial = cur | (1 << b)
        cnt = jnp.sum((bits < trial).astype(jnp.float32), axis=1,
                      keepdims=True)
        cur = jnp.where(cnt < float(_K), trial, cur)
    wk = jnp.where((bits <= cur) & inrad, 1.0, 0.0)
    padk = jnp.maximum(float(_K) - c_r, 0.0)

    colf = colg.astype(jnp.float32)
    m0 = jnp.min(d2, axis=1, keepdims=True)
    iselk0 = jnp.min(jnp.where(d2 == m0, colf, float(_N)), axis=1,
                     keepdims=True)
    wt = (wk + wb
          + padk * jnp.where(colf == iselk0, 1.0, 0.0)
          + padb * jnp.where(inball & (cs == 1.0), 1.0, 0.0))
    nsum = jnp.zeros((rows, _N), jnp.float32)
    for s in range(_SEQ):
        frs = f_ref[s]
        fts = ft_ref[s]
        f2r = jnp.sum(frs * frs, axis=1, keepdims=True)
        f2c = jnp.sum(fts * fts, axis=0, keepdims=True)
        fdot = jnp.dot(frs, fts, preferred_element_type=jnp.float32,
                       precision=jax.lax.Precision.DEFAULT)
        fd2 = f2r + f2c - 2.0 * fdot
        nsum = nsum + jnp.sqrt(jnp.maximum(fd2, 0.0))
    nsum = jnp.where(diag, 0.0, nsum)
    acc = jnp.sum(wt * nsum, axis=1, keepdims=True)

    @pl.when(i == 0)
    def _():
        out_ref[...] = jnp.zeros((1, 1), jnp.float32)
        wsum_ref[...] = jnp.zeros((1, 1), jnp.float32)

    out_ref[...] += jnp.sum(w * acc, axis=0, keepdims=True)
    wsum_ref[...] += jnp.sum(w, axis=0, keepdims=True)


@jax.jit
def kernel(pc_source, pred_flow, weights):
    p = pc_source[0]
    p4 = jnp.concatenate([p, weights[:, None]], axis=1)
    pt = p.T
    ft = jnp.transpose(pred_flow, (0, 2, 1))

    nb = _N // _ROWS
    total, wsum = pl.pallas_call(
        _tile_body,
        grid=(nb,),
        in_specs=[
            pl.BlockSpec((_ROWS, 4), lambda i: (i, 0)),
            pl.BlockSpec((3, _N), lambda i: (0, 0)),
            pl.BlockSpec((_SEQ, _ROWS, 3), lambda i: (0, i, 0)),
            pl.BlockSpec((_SEQ, 3, _N), lambda i: (0, 0, 0)),
        ],
        out_specs=[
            pl.BlockSpec((1, 1), lambda i: (0, 0)),
            pl.BlockSpec((1, 1), lambda i: (0, 0)),
        ],
        out_shape=[
            jax.ShapeDtypeStruct((1, 1), jnp.float32),
            jax.ShapeDtypeStruct((1, 1), jnp.float32),
        ],
        compiler_params=pltpu.CompilerParams(
            dimension_semantics=("arbitrary",),
        ),
    )(p4, pt, pred_flow, ft)

    t = total[0, 0]
    ws = wsum[0, 0]
    denom = jnp.where(ws > 0.0, ws, 1.0)
    return (t / denom) / (_K * _SEQ)


# speedup vs baseline: 1.5464x; 1.0066x over previous
import functools

import jax
import jax.numpy as jnp
from jax.experimental import pallas as pl
from jax.experimental.pallas import tpu as pltpu

_N = 4096
_SEQ = 8
_K = 32
_R_KNN2 = 0.25
_R_BQ2 = 0.5625
_BIG = 1e30
_ROWS = 64


def _tile_body(p4_ref, pt_ref, f_ref, ft_ref, out_ref, wsum_ref):
    i = pl.program_id(0)
    rows = _ROWS

    pr = p4_ref[:, 0:3]
    w = p4_ref[:, 3:4]
    pt = pt_ref[...]

    colg = jax.lax.broadcasted_iota(jnp.int32, (rows, _N), 1)
    rowg = i * rows + jax.lax.broadcasted_iota(jnp.int32, (rows, _N), 0)
    diag = colg == rowg

    p2r = jnp.sum(pr * pr, axis=1, keepdims=True)
    p2c = jnp.sum(pt * pt, axis=0, keepdims=True)
    dot = jnp.dot(pr, pt, preferred_element_type=jnp.float32,
                  precision=jax.lax.Precision.DEFAULT)
    d2 = jnp.maximum(p2r + p2c - 2.0 * dot, 0.0)

    inball = d2 < _R_BQ2
    ibf = inball.astype(jnp.float32)
    cs = ibf
    sh = 1
    while sh < _N:
        cs = cs + jnp.concatenate(
            [jnp.zeros((rows, sh), jnp.float32), cs[:, :-sh]], axis=1)
        sh *= 2
    wb = jnp.where(inball & (cs <= float(_K)), 1.0, 0.0)
    cnt = jnp.sum(ibf, axis=1, keepdims=True)
    padb = jnp.maximum(float(_K) - cnt, 0.0)

    inrad = d2 <= _R_KNN2
    bits = jax.lax.bitcast_convert_type(jnp.where(inrad, d2, _BIG),
                                        jnp.int32)
    c_r = jnp.sum(inrad.astype(jnp.float32), axis=1, keepdims=True)
    cur = jnp.zeros((rows, 1), jnp.int32)
    for b in range(30, -1, -1):
        trial = cur | (1 << b)
        cnt = jnp.sum((bits < trial).astype(jnp.float32), axis=1,
                      keepdims=True)
        cur = jnp.where(cnt < float(_K), trial, cur)
    wk = jnp.where((bits <= cur) & inrad, 1.0, 0.0)
    padk = jnp.maximum(float(_K) - c_r, 0.0)

    colf = colg.astype(jnp.float32)
    m0 = jnp.min(d2, axis=1, keepdims=True)
    iselk0 = jnp.min(jnp.where(d2 == m0, colf, float(_N)), axis=1,
                     keepdims=True)
    wt = (wk + wb
          + padk * jnp.where(colf == iselk0, 1.0, 0.0)
          + padb * jnp.where(inball & (cs == 1.0), 1.0, 0.0))
    nsum = jnp.zeros((rows, _N), jnp.float32)
    for s in range(_SEQ):
        frs = f_ref[s]
        fts = ft_ref[s]
        f2r = jnp.sum(frs * frs, axis=1, keepdims=True)
        f2c = jnp.sum(fts * fts, axis=0, keepdims=True)
        fdot = jnp.dot(frs, fts, preferred_element_type=jnp.float32,
                       precision=jax.lax.Precision.DEFAULT)
        fd2 = f2r + f2c - 2.0 * fdot
        nsum = nsum + jnp.sqrt(jnp.maximum(fd2, 0.0))
    nsum = jnp.where(diag, 0.0, nsum)
    acc = jnp.sum(wt * nsum, axis=1, keepdims=True)

    @pl.when(i == 0)
    def _():
        out_ref[...] = jnp.zeros((1, 1), jnp.float32)
        wsum_ref[...] = jnp.zeros((1, 1), jnp.float32)

    out_ref[...] += jnp.sum(w * acc, axis=0, keepdims=True)
    wsum_ref[...] += jnp.sum(w, axis=0, keepdims=True)


@jax.jit
def kernel(pc_source, pred_flow, weights):
    p = pc_source[0]
    p4 = jnp.concatenate([p, weights[:, None]], axis=1)
    pt = p.T
    ft = jnp.transpose(pred_flow, (0, 2, 1))

    nb = _N // _ROWS
    total, wsum = pl.pallas_call(
        _tile_body,
        grid=(nb,),
        in_specs=[
            pl.BlockSpec((_ROWS, 4), lambda i: (i, 0)),
            pl.BlockSpec((3, _N), lambda i: (0, 0)),
            pl.BlockSpec((_SEQ, _ROWS, 3), lambda i: (0, i, 0)),
            pl.BlockSpec((_SEQ, 3, _N), lambda i: (0, 0, 0)),
        ],
        out_specs=[
            pl.BlockSpec((1, 1), lambda i: (0, 0)),
            pl.BlockSpec((1, 1), lambda i: (0, 0)),
        ],
        out_shape=[
            jax.ShapeDtypeStruct((1, 1), jnp.float32),
            jax.ShapeDtypeStruct((1, 1), jnp.float32),
        ],
        compiler_params=pltpu.CompilerParams(
            dimension_semantics=("arbitrary",),
        ),
    )(p4, pt, pred_flow, ft)

    t = total[0, 0]
    ws = wsum[0, 0]
    denom = jnp.where(ws > 0.0, ws, 1.0)
    return (t / denom) / (_K * _SEQ)
